# baseline (speedup 1.0000x reference)
import functools

import jax
import jax.numpy as jnp
from jax import lax
from jax.experimental import pallas as pl
from jax.experimental.pallas import tpu as pltpu
from jax.experimental.pallas import tpu_sc as plsc

EMBED = 32
BATCH = 16384
NUM_SLABS = 1000000 // 8
NUM_TILES = 1000000 // 32

_info = plsc.get_sparse_core_info()
_NC, _NS = _info.num_cores, _info.num_subcores
_NW = _NC * _NS
_BPW = BATCH // _NW
_CHU = 32
_CHI = 32


def _sc_gather(utp, it3, uent, uoff, islab, ilane):
    mesh = plsc.VectorSubcoreMesh(core_axis_name="c", subcore_axis_name="s")

    @functools.partial(
        pl.kernel,
        mesh=mesh,
        compiler_params=pltpu.CompilerParams(needs_layout_passes=False),
        out_type=[
            jax.ShapeDtypeStruct((BATCH // 4, 128), jnp.float32),
            jax.ShapeDtypeStruct((BATCH // 4, 128), jnp.float32),
        ],
        scratch_types=[
            pltpu.VMEM((_BPW,), jnp.int32),
            pltpu.VMEM((_BPW,), jnp.int32),
            pltpu.VMEM((_BPW,), jnp.int32),
            pltpu.VMEM((_BPW,), jnp.int32),
            pltpu.VMEM((2, _CHU, 8, 128), jnp.float32),
            pltpu.VMEM((_CHI, 8, EMBED), jnp.float32),
            pltpu.VMEM((2, _CHU // 4, 128), jnp.float32),
            pltpu.SemaphoreType.DMA,
            pltpu.SemaphoreType.DMA,
            pltpu.SemaphoreType.DMA,
            pltpu.SemaphoreType.DMA,
        ],
    )
    def k(ut_hbm, it_hbm, ue_hbm, uo_hbm, is_hbm, il_hbm, gu_out, gi_out,
          uent_v, uoff_v, islab_v, ilane_v, uring, ibuf, obuf,
          gsem0, gsem1, wsem0, wsem1):
        gsems = [gsem0, gsem1]
        wsems = [wsem0, wsem1]
        wid = lax.axis_index("s") * _NC + lax.axis_index("c")
        base = wid * _BPW
        pltpu.sync_copy(ue_hbm.at[pl.ds(base, _BPW)], uent_v)
        pltpu.sync_copy(uo_hbm.at[pl.ds(base, _BPW)], uoff_v)
        pltpu.sync_copy(is_hbm.at[pl.ds(base, _BPW)], islab_v)
        pltpu.sync_copy(il_hbm.at[pl.ds(base, _BPW)], ilane_v)
        iota16 = lax.iota(jnp.int32, 16)

        def fire(tab, ids_v, dst, ch, c, gsem):
            def body(g, carry):
                id16 = ids_v[pl.ds(c * ch + g * 16, 16)]
                for r in range(16):
                    sl = jnp.max(jnp.where(iota16 == r, id16, 0))
                    pltpu.async_copy(tab.at[sl], dst.at[g * 16 + r], gsem)
                return carry

            lax.fori_loop(0, ch // 16, body, 0)

        def drain_gather(tab, dst, gsem):
            pltpu.make_async_copy(
                tab.at[pl.ds(0, dst.shape[0])], dst, gsem).wait()

        def extract(buf, off_v, wide, ch, c, oslot):
            def grp(g, carry):
                i16 = g * 16 + iota16
                off16 = off_v[pl.ds(c * ch + g * 16, 16)]
                if wide:
                    row16 = off16 & 7
                    col0 = lax.shift_right_logical(off16, 3) * EMBED
                else:
                    row16 = off16
                    col0 = iota16 * 0
                orow16 = lax.shift_right_logical(i16, 2)
                ocol0 = (i16 & 3) * EMBED
                for j in range(EMBED):
                    w = plsc.load_gather(buf, [i16, row16, col0 + j])
                    plsc.store_scatter(obuf.at[oslot], [orow16, ocol0 + j], w)
                return carry

            lax.fori_loop(0, ch // 16, grp, 0)

        def writeback(out, ch, c, oslot, wsem):
            pos = pl.multiple_of((base + c * ch) // 4, ch // 4)
            return pltpu.async_copy(
                obuf.at[oslot], out.at[pl.ds(pos, ch // 4)], wsem)

        def drain_wb(out, ch, oslot, wsem):
            pltpu.make_async_copy(
                out.at[pl.ds(0, ch // 4)], obuf.at[oslot], wsem).wait()

        def upair(p, carry):
            for slot in range(2):
                fire(ut_hbm, uent_v, uring.at[slot], _CHU, p * 2 + slot,
                     gsems[slot])
            for slot in range(2):
                c = p * 2 + slot
                drain_gather(ut_hbm, uring.at[slot], gsems[slot])

                @pl.when(p > 0)
                def _():
                    drain_wb(gu_out, _CHU, slot, wsems[slot])

                extract(uring.at[slot], uoff_v, True, _CHU, c, slot)
                writeback(gu_out, _CHU, c, slot, wsems[slot])
            return carry

        lax.fori_loop(0, _BPW // _CHU // 2, upair, 0)
        drain_wb(gu_out, _CHU, 0, wsems[0])
        drain_wb(gu_out, _CHU, 1, wsems[1])

        def istep(c, carry):
            fire(it_hbm, islab_v, ibuf, _CHI, c, gsems[0])
            drain_gather(it_hbm, ibuf, gsems[0])

            @pl.when(c > 0)
            def _():
                drain_wb(gi_out, _CHI, 0, wsems[0])

            extract(ibuf, ilane_v, False, _CHI, c, 0)
            writeback(gi_out, _CHI, c, 0, wsems[0])
            return carry

        lax.fori_loop(0, _BPW // _CHI, istep, 0)
        drain_wb(gi_out, _CHI, 0, wsems[0])

    return k(utp, it3, uent, uoff, islab, ilane)


_TCOLS = 2048


def _repack_body(xt_ref, eye_ref, out_ref):
    x = xt_ref[...]
    y = jax.lax.dot_general(x, eye_ref[...], (((0,), (0,)), ((), ())),
                            preferred_element_type=jnp.float32)
    y4 = y.reshape(_TCOLS // EMBED, 4, 8, EMBED)
    out_ref[...] = jnp.concatenate(
        [y4[:, k, :, :] for k in range(4)], axis=-1)


def _tc_repack(table_t):
    grid = (pl.cdiv(1000000, _TCOLS),)
    eye = jnp.eye(EMBED, dtype=jnp.float32)
    return pl.pallas_call(
        _repack_body,
        grid=grid,
        in_specs=[
            pl.BlockSpec((EMBED, _TCOLS), lambda c: (0, c)),
            pl.BlockSpec((EMBED, EMBED), lambda c: (0, 0)),
        ],
        out_specs=pl.BlockSpec((_TCOLS // EMBED, 8, 128), lambda c: (c, 0, 0)),
        out_shape=jax.ShapeDtypeStruct((NUM_TILES, 8, 128), jnp.float32),
    )(table_t, eye)


def _mlp_body(ue_ref, ie_ref, w1u_ref, w1i_ref, b1_ref, w2_ref, b2_ref,
              w3_ref, b3_ref, out_ref):
    u = ue_ref[...]
    v = ie_ref[...]
    h = (jnp.dot(u, w1u_ref[...], preferred_element_type=jnp.float32)
         + jnp.dot(v, w1i_ref[...], preferred_element_type=jnp.float32)
         + b1_ref[...])
    h = jnp.maximum(h, 0.0)
    h2 = jnp.dot(h, w2_ref[...], preferred_element_type=jnp.float32) + b2_ref[...]
    h2 = jnp.maximum(h2, 0.0)
    z = jnp.sum(h2 * w3_ref[...], axis=1) + b3_ref[0, 0]
    out_ref[...] = 1.0 / (1.0 + jnp.exp(-z))


def _tc_mlp(ue, ie, w1u, w1i, b1r, w2t, b2r, w3r, b3r):
    blk = 2048
    grid = (BATCH // blk,)
    full = lambda shape: pl.BlockSpec(shape, lambda i: (0,) * len(shape))
    return pl.pallas_call(
        _mlp_body,
        grid=grid,
        in_specs=[
            pl.BlockSpec((blk, EMBED), lambda i: (i, 0)),
            pl.BlockSpec((blk, EMBED), lambda i: (i, 0)),
            full((EMBED, 64)),
            full((EMBED, 64)),
            full((1, 64)),
            full((64, EMBED)),
            full((1, EMBED)),
            full((1, EMBED)),
            full((1, 1)),
        ],
        out_specs=pl.BlockSpec((blk,), lambda i: (i,)),
        out_shape=jax.ShapeDtypeStruct((BATCH,), jnp.float32),
    )(ue, ie, w1u, w1i, b1r, w2t, b2r, w3r, b3r)


def kernel(batch_data, user_table, item_table, W1, b1, W2, b2, W3, b3):
    uidx = batch_data[:, 0]
    iidx = batch_data[:, 1]
    utp = _tc_repack(user_table.T)
    it3 = item_table.reshape(NUM_SLABS, 8, EMBED)
    uent = uidx // 32
    uoff = uidx % 32
    islab = iidx // 8
    ilane = iidx % 8
    pu, pi = _sc_gather(utp, it3, uent, uoff, islab, ilane)
    ue = pu.reshape(BATCH, EMBED)
    ie = pi.reshape(BATCH, EMBED)
    w1t = W1.T
    return _tc_mlp(ue, ie, w1t[:EMBED], w1t[EMBED:], b1.reshape(1, 64),
                   W2.T, b2.reshape(1, EMBED), W3, b3.reshape(1, 1))


# speedup vs baseline: 1.3960x; 1.3960x over previous
import functools

import jax
import jax.numpy as jnp
from jax import lax
from jax.experimental import pallas as pl
from jax.experimental.pallas import tpu as pltpu
from jax.experimental.pallas import tpu_sc as plsc

EMBED = 32
BATCH = 16384
NUM_SLABS = 1000000 // 8

_info = plsc.get_sparse_core_info()
_NC, _NS = _info.num_cores, _info.num_subcores
_NW = _NC * _NS
_BPW = BATCH // _NW
_CH = 32
_NSTEP = _BPW // _CH


def _sc_gather(ut3, it3, uslab, ulane, islab, ilane):
    mesh = plsc.VectorSubcoreMesh(core_axis_name="c", subcore_axis_name="s")

    @functools.partial(
        pl.kernel,
        mesh=mesh,
        compiler_params=pltpu.CompilerParams(needs_layout_passes=False),
        out_type=[
            jax.ShapeDtypeStruct((BATCH // 4, 128), jnp.float32),
            jax.ShapeDtypeStruct((BATCH // 4, 128), jnp.float32),
        ],
        scratch_types=[
            pltpu.VMEM((_BPW,), jnp.int32),
            pltpu.VMEM((_BPW,), jnp.int32),
            pltpu.VMEM((_BPW,), jnp.int32),
            pltpu.VMEM((_BPW,), jnp.int32),
            pltpu.VMEM((2, _CH, 8, EMBED), jnp.float32),
            pltpu.VMEM((2, _CH // 4, 128), jnp.float32),
            pltpu.SemaphoreType.DMA,
            pltpu.SemaphoreType.DMA,
            pltpu.SemaphoreType.DMA,
        ],
    )
    def k(ut_hbm, it_hbm, us_hbm, ul_hbm, is_hbm, il_hbm, gu_out, gi_out,
          ulane_v, ilane_v, us_v, is_v, ring, obuf, sem0, sem1, wsem):
        sems = [sem0, sem1]
        wid = lax.axis_index("s") * _NC + lax.axis_index("c")
        base = wid * _BPW
        pltpu.sync_copy(us_hbm.at[pl.ds(base, _BPW)], us_v)
        pltpu.sync_copy(is_hbm.at[pl.ds(base, _BPW)], is_v)
        pltpu.sync_copy(ul_hbm.at[pl.ds(base, _BPW)], ulane_v)
        pltpu.sync_copy(il_hbm.at[pl.ds(base, _BPW)], ilane_v)
        iota16 = lax.iota(jnp.int32, 16)

        def fire(s):
            tab, slabs = (ut_hbm, us_v) if s < _NSTEP else (it_hbm, is_v)
            c = s % _NSTEP
            slot = s % 2

            def body(g, carry):
                slab16 = slabs[pl.ds(c * _CH + g * 16, 16)]
                for r in range(16):
                    sl = jnp.max(jnp.where(iota16 == r, slab16, 0))
                    pltpu.async_copy(tab.at[sl], ring.at[slot, g * 16 + r],
                                     sems[slot])
                return carry

            lax.fori_loop(0, _CH // 16, body, 0)

        def drain(s):
            pltpu.make_async_copy(
                ut_hbm.at[pl.ds(0, _CH)], ring.at[s % 2], sems[s % 2]).wait()

        def extract(s):
            lanes = ulane_v if s < _NSTEP else ilane_v
            c = s % _NSTEP
            slot = s % 2

            def grp(g, carry):
                i16 = g * 16 + iota16
                lane16 = lanes[pl.ds(c * _CH + g * 16, 16)]
                orow16 = lax.shift_right_logical(i16, 2)
                ocol0 = (i16 & 3) * EMBED
                for j in range(EMBED):
                    w = plsc.load_gather(
                        ring.at[slot], [i16, lane16, iota16 * 0 + j])
                    plsc.store_scatter(
                        obuf.at[slot], [orow16, ocol0 + j], w)
                return carry

            lax.fori_loop(0, _CH // 16, grp, 0)

        def writeback(s):
            out = gu_out if s < _NSTEP else gi_out
            c = s % _NSTEP
            pos = pl.multiple_of((base + c * _CH) // 4, _CH // 4)
            return pltpu.async_copy(
                obuf.at[s % 2], out.at[pl.ds(pos, _CH // 4)], wsem)

        wbs = {}
        fire(0)
        for s in range(2 * _NSTEP):
            if s + 1 < 2 * _NSTEP:
                fire(s + 1)
            drain(s)
            if s >= 2:
                wbs[s - 2].wait()
            extract(s)
            wbs[s] = writeback(s)
        wbs[2 * _NSTEP - 2].wait()
        wbs[2 * _NSTEP - 1].wait()

    return k(ut3, it3, uslab, ulane, islab, ilane)


def _mlp_body(ue_ref, ie_ref, w1u_ref, w1i_ref, b1_ref, w2_ref, b2_ref,
              w3_ref, b3_ref, out_ref):
    u = ue_ref[...]
    v = ie_ref[...]
    h = (jnp.dot(u, w1u_ref[...], preferred_element_type=jnp.float32)
         + jnp.dot(v, w1i_ref[...], preferred_element_type=jnp.float32)
         + b1_ref[...])
    h = jnp.maximum(h, 0.0)
    h2 = jnp.dot(h, w2_ref[...], preferred_element_type=jnp.float32) + b2_ref[...]
    h2 = jnp.maximum(h2, 0.0)
    z = jnp.sum(h2 * w3_ref[...], axis=1) + b3_ref[0, 0]
    out_ref[...] = 1.0 / (1.0 + jnp.exp(-z))


def _tc_mlp(ue, ie, w1u, w1i, b1r, w2t, b2r, w3r, b3r):
    blk = 2048
    grid = (BATCH // blk,)
    full = lambda shape: pl.BlockSpec(shape, lambda i: (0,) * len(shape))
    return pl.pallas_call(
        _mlp_body,
        grid=grid,
        in_specs=[
            pl.BlockSpec((blk, EMBED), lambda i: (i, 0)),
            pl.BlockSpec((blk, EMBED), lambda i: (i, 0)),
            full((EMBED, 64)),
            full((EMBED, 64)),
            full((1, 64)),
            full((64, EMBED)),
            full((1, EMBED)),
            full((1, EMBED)),
            full((1, 1)),
        ],
        out_specs=pl.BlockSpec((blk,), lambda i: (i,)),
        out_shape=jax.ShapeDtypeStruct((BATCH,), jnp.float32),
    )(ue, ie, w1u, w1i, b1r, w2t, b2r, w3r, b3r)


def kernel(batch_data, user_table, item_table, W1, b1, W2, b2, W3, b3):
    uidx = batch_data[:, 0]
    iidx = batch_data[:, 1]
    ut3 = user_table.reshape(NUM_SLABS, 8, EMBED)
    it3 = item_table.reshape(NUM_SLABS, 8, EMBED)
    uslab = uidx // 8
    islab = iidx // 8
    ulane = uidx % 8
    ilane = iidx % 8
    pu, pi = _sc_gather(ut3, it3, uslab, ulane, islab, ilane)
    ue = pu.reshape(BATCH, EMBED)
    ie = pi.reshape(BATCH, EMBED)
    w1t = W1.T
    return _tc_mlp(ue, ie, w1t[:EMBED], w1t[EMBED:], b1.reshape(1, 64),
                   W2.T, b2.reshape(1, EMBED), W3, b3.reshape(1, 1))


# speedup vs baseline: 1.4862x; 1.0646x over previous
import functools

import jax
import jax.numpy as jnp
from jax import lax
from jax.experimental import pallas as pl
from jax.experimental.pallas import tpu as pltpu
from jax.experimental.pallas import tpu_sc as plsc

EMBED = 32
BATCH = 16384
NUM_SLABS = 1000000 // 8

_info = plsc.get_sparse_core_info()
_NC, _NS = _info.num_cores, _info.num_subcores
_NW = _NC * _NS
_BPW = BATCH // _NW
_CH = 32
_NSTEP = _BPW // _CH


def _sc_gather(ut3, it3, uslab, ulane, islab, ilane):
    mesh = plsc.VectorSubcoreMesh(core_axis_name="c", subcore_axis_name="s")

    @functools.partial(
        pl.kernel,
        mesh=mesh,
        compiler_params=pltpu.CompilerParams(needs_layout_passes=False),
        out_type=[
            jax.ShapeDtypeStruct((BATCH // 4, 128), jnp.float32),
            jax.ShapeDtypeStruct((BATCH // 4, 128), jnp.float32),
        ],
        scratch_types=[
            pltpu.VMEM((_BPW,), jnp.int32),
            pltpu.VMEM((_BPW,), jnp.int32),
            pltpu.VMEM((_BPW,), jnp.int32),
            pltpu.VMEM((_BPW,), jnp.int32),
            pltpu.VMEM((2, _CH, 8, EMBED), jnp.float32),
            pltpu.VMEM((2, _CH // 4, 128), jnp.float32),
            pltpu.SemaphoreType.DMA,
            pltpu.SemaphoreType.DMA,
            pltpu.SemaphoreType.DMA,
        ],
    )
    def k(ut_hbm, it_hbm, us_hbm, ul_hbm, is_hbm, il_hbm, gu_out, gi_out,
          ulane_v, ilane_v, us_v, is_v, ring, obuf, sem0, sem1, wsem):
        sems = [sem0, sem1]
        wid = lax.axis_index("s") * _NC + lax.axis_index("c")
        base = wid * _BPW
        pltpu.sync_copy(us_hbm.at[pl.ds(base, _BPW)], us_v)
        pltpu.sync_copy(is_hbm.at[pl.ds(base, _BPW)], is_v)
        pltpu.sync_copy(ul_hbm.at[pl.ds(base, _BPW)], ulane_v)
        pltpu.sync_copy(il_hbm.at[pl.ds(base, _BPW)], ilane_v)
        iota16 = lax.iota(jnp.int32, 16)

        def fire(s):
            tab, slabs = (ut_hbm, us_v) if s < _NSTEP else (it_hbm, is_v)
            c = s % _NSTEP
            slot = s % 2

            def body(g, carry):
                slab16 = slabs[pl.ds(c * _CH + g * 16, 16)]
                for r in range(16):
                    sl = jnp.max(jnp.where(iota16 == r, slab16, 0))
                    pltpu.async_copy(tab.at[sl], ring.at[slot, g * 16 + r],
                                     sems[slot])
                return carry

            lax.fori_loop(0, _CH // 16, body, 0)

        def drain(s):
            pltpu.make_async_copy(
                ut_hbm.at[pl.ds(0, _CH)], ring.at[s % 2], sems[s % 2]).wait()

        def extract(s):
            lanes = ulane_v if s < _NSTEP else ilane_v
            c = s % _NSTEP
            slot = s % 2

            def grp(g, carry):
                i16 = g * 16 + iota16
                lane16 = lanes[pl.ds(c * _CH + g * 16, 16)]
                orow16 = lax.shift_right_logical(i16, 2)
                ocol0 = (i16 & 3) * EMBED
                for j in range(EMBED):
                    w = plsc.load_gather(
                        ring.at[slot], [i16, lane16, iota16 * 0 + j])
                    plsc.store_scatter(
                        obuf.at[slot], [orow16, ocol0 + j], w)
                return carry

            lax.fori_loop(0, _CH // 16, grp, 0)

        def writeback(s):
            out = gu_out if s < _NSTEP else gi_out
            c = s % _NSTEP
            pos = pl.multiple_of((base + c * _CH) // 4, _CH // 4)
            return pltpu.async_copy(
                obuf.at[s % 2], out.at[pl.ds(pos, _CH // 4)], wsem)

        wbs = {}
        fire(0)
        for s in range(2 * _NSTEP):
            if s + 1 < 2 * _NSTEP:
                fire(s + 1)
            drain(s)
            if s >= 2:
                wbs[s - 2].wait()
            extract(s)
            wbs[s] = writeback(s)
        wbs[2 * _NSTEP - 2].wait()
        wbs[2 * _NSTEP - 1].wait()

    return k(ut3, it3, uslab, ulane, islab, ilane)


def _mlp_body(ue_ref, ie_ref, w1u_ref, w1i_ref, b1_ref, w2_ref, b2_ref,
              w3_ref, b3_ref, out_ref):
    u = ue_ref[...]
    v = ie_ref[...]
    h = (jnp.dot(u, w1u_ref[...], preferred_element_type=jnp.float32)
         + jnp.dot(v, w1i_ref[...], preferred_element_type=jnp.float32)
         + b1_ref[...])
    h = jnp.maximum(h, 0.0)
    h2 = jnp.dot(h, w2_ref[...], preferred_element_type=jnp.float32) + b2_ref[...]
    h2 = jnp.maximum(h2, 0.0)
    z = jnp.dot(h2, w3_ref[...], preferred_element_type=jnp.float32) + b3_ref[0, 0]
    out_ref[...] = 1.0 / (1.0 + jnp.exp(-z))


def _tc_mlp(pu, pi, w1u, w1i, b1r, w2bd, b2r, w3bd, b3r):
    blk = 512
    grid = (BATCH // 4 // blk,)
    full = lambda shape: pl.BlockSpec(shape, lambda i: (0,) * len(shape))
    return pl.pallas_call(
        _mlp_body,
        grid=grid,
        in_specs=[
            pl.BlockSpec((blk, 128), lambda i: (i, 0)),
            pl.BlockSpec((blk, 128), lambda i: (i, 0)),
            full((128, 256)),
            full((128, 256)),
            full((1, 256)),
            full((256, 128)),
            full((1, 128)),
            full((128, 4)),
            full((1, 1)),
        ],
        out_specs=pl.BlockSpec((blk, 4), lambda i: (i, 0)),
        out_shape=jax.ShapeDtypeStruct((BATCH // 4, 4), jnp.float32),
    )(pu, pi, w1u, w1i, b1r, w2bd, b2r, w3bd, b3r)


def _block_diag4(m):
    return jax.scipy.linalg.block_diag(m, m, m, m)


def kernel(batch_data, user_table, item_table, W1, b1, W2, b2, W3, b3):
    uidx = batch_data[:, 0]
    iidx = batch_data[:, 1]
    ut3 = user_table.reshape(NUM_SLABS, 8, EMBED)
    it3 = item_table.reshape(NUM_SLABS, 8, EMBED)
    uslab = uidx // 8
    islab = iidx // 8
    ulane = uidx % 8
    ilane = iidx % 8
    pu, pi = _sc_gather(ut3, it3, uslab, ulane, islab, ilane)
    w1t = W1.T
    w1ubd = _block_diag4(w1t[:EMBED])
    w1ibd = _block_diag4(w1t[EMBED:])
    w2bd = _block_diag4(W2.T)
    w3bd = _block_diag4(W3.T)
    b1bd = jnp.tile(b1, 4).reshape(1, 256)
    b2bd = jnp.tile(b2, 4).reshape(1, 128)
    out4 = _tc_mlp(pu, pi, w1ubd, w1ibd, b1bd, w2bd, b2bd, w3bd,
                   b3.reshape(1, 1))
    return out4.reshape(BATCH)
